# baseline (speedup 1.0000x reference)
import functools

import jax
import jax.numpy as jnp
from jax import lax
from jax.experimental import pallas as pl
from jax.experimental.pallas import tpu as pltpu
from jax.experimental.pallas import tpu_sc as plsc

B = 1
S = 2048
D = 1024
H = 8
HD = D // H
T = S * H
E = 8
K = 2
F = 512

BM = 256
NBS = T * K // BM + E
PS = NBS * BM
NW = 32
TPW = T // NW
CH = 128
NCH = TPW // CH


def _mm_bias_kernel(x_ref, w_ref, b_ref, o_ref):
    o_ref[...] = (
        jnp.dot(x_ref[...], w_ref[...], preferred_element_type=jnp.float32)
        + b_ref[...]
    )


def _matmul_bias(x, w, b, bm=512):
    M, Kd = x.shape
    N = w.shape[1]
    return pl.pallas_call(
        _mm_bias_kernel,
        grid=(M // bm,),
        in_specs=[
            pl.BlockSpec((bm, Kd), lambda i: (i, 0)),
            pl.BlockSpec((Kd, N), lambda i: (0, 0)),
            pl.BlockSpec((1, N), lambda i: (0, 0)),
        ],
        out_specs=pl.BlockSpec((bm, N), lambda i: (i, 0)),
        out_shape=jax.ShapeDtypeStruct((M, N), jnp.float32),
    )(x, w, b.reshape(1, N))


def _router_kernel(y_ref, wg_ref, e0_ref, e1_ref, g0_ref, g1_ref):
    y = y_ref[...]
    logits = jnp.dot(y, wg_ref[...], preferred_element_type=jnp.float32)
    probs = jax.nn.softmax(logits, axis=-1)
    iota = jax.lax.broadcasted_iota(jnp.int32, probs.shape, 1)
    m1 = jnp.max(probs, axis=-1, keepdims=True)
    i1 = jnp.min(jnp.where(probs == m1, iota, E), axis=-1, keepdims=True)
    pm = jnp.where(iota == i1, -jnp.inf, probs)
    m2 = jnp.max(pm, axis=-1, keepdims=True)
    i2 = jnp.min(jnp.where(pm == m2, iota, E), axis=-1, keepdims=True)
    denom = m1 + m2 + 1e-9
    e0_ref[...] = i1
    e1_ref[...] = i2
    g0_ref[...] = m1 / denom
    g1_ref[...] = m2 / denom


def _router(y16, Wg, bm=4096):
    return pl.pallas_call(
        _router_kernel,
        grid=(T // bm,),
        in_specs=[
            pl.BlockSpec((bm, HD), lambda i: (i, 0)),
            pl.BlockSpec((HD, E), lambda i: (0, 0)),
        ],
        out_specs=[
            pl.BlockSpec((bm, 1), lambda i: (i, 0)),
            pl.BlockSpec((bm, 1), lambda i: (i, 0)),
            pl.BlockSpec((bm, 1), lambda i: (i, 0)),
            pl.BlockSpec((bm, 1), lambda i: (i, 0)),
        ],
        out_shape=[
            jax.ShapeDtypeStruct((T, 1), jnp.int32),
            jax.ShapeDtypeStruct((T, 1), jnp.int32),
            jax.ShapeDtypeStruct((T, 1), jnp.float32),
            jax.ShapeDtypeStruct((T, 1), jnp.float32),
        ],
    )(y16, Wg)


def _sortmeta_kernel(e0_ref, e1_ref, d0_ref, d1_ref, be_ref):
    f32 = jnp.float32
    ir = jax.lax.broadcasted_iota(jnp.int32, (128, 128), 0)
    ic = jax.lax.broadcasted_iota(jnp.int32, (128, 128), 1)
    U = (ir < ic).astype(f32)
    lr = jax.lax.broadcasted_iota(jnp.int32, (16, 16), 0)
    lc = jax.lax.broadcasted_iota(jnp.int32, (16, 16), 1)
    L16 = (lr > lc).astype(f32)

    e0 = e0_ref[...]
    e1 = e1_ref[...]

    counts = []
    for e in range(E):
        m = (e0 == e).astype(f32) + (e1 == e).astype(f32)
        counts.append(jnp.sum(m))
    starts = []
    s = jnp.float32(0.0)
    for e in range(E):
        starts.append(s)
        s = s + jnp.ceil(counts[e] / BM) * BM

    ib = jax.lax.broadcasted_iota(jnp.int32, (1, 256), 1).astype(f32) * BM
    be = jnp.zeros((1, 256), jnp.int32)
    for e in range(E):
        be = be + (ib >= starts[e]).astype(jnp.int32)
    be_ref[...] = be - 1

    carry = [jnp.float32(0.0)] * E
    for c in range(8):
        sl = slice(16 * c, 16 * c + 16)
        e0c = e0[sl, :]
        e1c = e1[sl, :]
        d0c = jnp.zeros((16, 128), f32)
        d1c = jnp.zeros((16, 128), f32)
        for e in range(E):
            m = (e0c == e).astype(f32) + (e1c == e).astype(f32)
            lane_excl = jnp.dot(m, U, preferred_element_type=f32)
            rowtot = jnp.sum(m, axis=1, keepdims=True)
            row_excl = jnp.dot(L16, rowtot, preferred_element_type=f32)
            slot = starts[e] + carry[e] + row_excl + lane_excl
            d0c = d0c + jnp.where(e0c == e, slot, 0.0)
            d1c = d1c + jnp.where(e1c == e, slot, 0.0)
            carry[e] = carry[e] + jnp.sum(m)
        d0_ref[sl, :] = d0c.astype(jnp.int32)
        d1_ref[sl, :] = d1c.astype(jnp.int32)


def _sortmeta(e0, e1):
    return pl.pallas_call(
        _sortmeta_kernel,
        grid=(1,),
        in_specs=[
            pl.BlockSpec((128, 128), lambda i: (0, 0)),
            pl.BlockSpec((128, 128), lambda i: (0, 0)),
        ],
        out_specs=[
            pl.BlockSpec((128, 128), lambda i: (0, 0)),
            pl.BlockSpec((128, 128), lambda i: (0, 0)),
            pl.BlockSpec((1, 256), lambda i: (0, 0)),
        ],
        out_shape=[
            jax.ShapeDtypeStruct((128, 128), jnp.int32),
            jax.ShapeDtypeStruct((128, 128), jnp.int32),
            jax.ShapeDtypeStruct((1, 256), jnp.int32),
        ],
    )(e0.reshape(128, 128), e1.reshape(128, 128))


def _dispatch_body(y_hbm, d0_hbm, d1_hbm, g0_hbm, g1_hbm, yg_hbm, gyg_hbm,
                   ybuf, d0b, d1b, g0b, g1b, sem):
    wid = lax.axis_index("s") * 2 + lax.axis_index("c")
    base = wid * TPW
    pltpu.sync_copy(y_hbm.at[pl.ds(base, TPW)], ybuf)
    pltpu.sync_copy(d0_hbm.at[wid], d0b)
    pltpu.sync_copy(d1_hbm.at[wid], d1b)
    pltpu.sync_copy(g0_hbm.at[wid], g0b)
    pltpu.sync_copy(g1_hbm.at[wid], g1b)
    for j in range(NCH):
        rows = ybuf.at[pl.ds(j * CH, CH)]
        pltpu.async_copy(rows, yg_hbm.at[d0b.at[j]], sem).wait()
        pltpu.async_copy(rows, yg_hbm.at[d1b.at[j]], sem).wait()
        pltpu.async_copy(g0b.at[j], gyg_hbm.at[d0b.at[j]], sem).wait()
        pltpu.async_copy(g1b.at[j], gyg_hbm.at[d1b.at[j]], sem).wait()


def _dispatch(y16, d0, d1, g0, g1):
    mesh = plsc.VectorSubcoreMesh(core_axis_name="c", subcore_axis_name="s")
    kfn = functools.partial(
        pl.kernel,
        out_type=[
            jax.ShapeDtypeStruct((PS, HD), jnp.float32),
            jax.ShapeDtypeStruct((PS,), jnp.float32),
        ],
        mesh=mesh,
        scratch_types=[
            pltpu.VMEM((TPW, HD), jnp.float32),
            pltpu.VMEM((NCH, CH), jnp.int32),
            pltpu.VMEM((NCH, CH), jnp.int32),
            pltpu.VMEM((NCH, CH), jnp.float32),
            pltpu.VMEM((NCH, CH), jnp.float32),
            pltpu.SemaphoreType.DMA,
        ],
    )(_dispatch_body)
    return kfn(
        y16,
        d0.reshape(NW, NCH, CH),
        d1.reshape(NW, NCH, CH),
        g0.reshape(NW, NCH, CH),
        g1.reshape(NW, NCH, CH),
    )


def _ffn_kernel(be_ref, yg_ref, w1_ref, b1_ref, w2_ref, b2_ref, g_ref, o_ref):
    ygb = yg_ref[...]
    h = jax.nn.gelu(
        jnp.dot(ygb, w1_ref[0], preferred_element_type=jnp.float32)
        + b1_ref[0]
    )
    eo = jnp.dot(h, w2_ref[0], preferred_element_type=jnp.float32) + b2_ref[0]
    o_ref[...] = eo * g_ref[...]


def _grouped_ffn(be, yg, gyg, W1, b1, W2, b2):
    grid_spec = pltpu.PrefetchScalarGridSpec(
        num_scalar_prefetch=1,
        grid=(NBS,),
        in_specs=[
            pl.BlockSpec((BM, HD), lambda i, be: (i, 0)),
            pl.BlockSpec((1, HD, F), lambda i, be: (be[0, i], 0, 0)),
            pl.BlockSpec((1, 1, F), lambda i, be: (be[0, i], 0, 0)),
            pl.BlockSpec((1, F, HD), lambda i, be: (be[0, i], 0, 0)),
            pl.BlockSpec((1, 1, HD), lambda i, be: (be[0, i], 0, 0)),
            pl.BlockSpec((BM, 1), lambda i, be: (i, 0)),
        ],
        out_specs=pl.BlockSpec((BM, HD), lambda i, be: (i, 0)),
    )
    return pl.pallas_call(
        _ffn_kernel,
        grid_spec=grid_spec,
        out_shape=jax.ShapeDtypeStruct((PS, HD), jnp.float32),
    )(be, yg, W1, b1.reshape(E, 1, F), W2, b2.reshape(E, 1, HD),
      gyg.reshape(PS, 1))


def _combine_body(eo_hbm, d0_hbm, d1_hbm, ymoe_hbm,
                  d0b, d1b, r0, r1, ob, sem):
    wid = lax.axis_index("s") * 2 + lax.axis_index("c")
    base = wid * TPW
    pltpu.sync_copy(d0_hbm.at[wid], d0b)
    pltpu.sync_copy(d1_hbm.at[wid], d1b)
    for j in range(NCH):
        pltpu.async_copy(eo_hbm.at[d0b.at[j]], r0, sem).wait()
        pltpu.async_copy(eo_hbm.at[d1b.at[j]], r1, sem).wait()

        def body(r, _):
            for c in range(HD // 16):
                csl = pl.ds(c * 16, 16)
                ob[r, csl] = r0[r, csl] + r1[r, csl]
            return 0

        lax.fori_loop(0, CH, body, 0)
        pltpu.sync_copy(ob, ymoe_hbm.at[pl.ds(base + j * CH, CH)])


def _combine(eo, d0, d1):
    mesh = plsc.VectorSubcoreMesh(core_axis_name="c", subcore_axis_name="s")
    kfn = functools.partial(
        pl.kernel,
        out_type=jax.ShapeDtypeStruct((T, HD), jnp.float32),
        mesh=mesh,
        scratch_types=[
            pltpu.VMEM((NCH, CH), jnp.int32),
            pltpu.VMEM((NCH, CH), jnp.int32),
            pltpu.VMEM((CH, HD), jnp.float32),
            pltpu.VMEM((CH, HD), jnp.float32),
            pltpu.VMEM((CH, HD), jnp.float32),
            pltpu.SemaphoreType.DMA,
        ],
    )(_combine_body)
    return kfn(eo, d0.reshape(NW, NCH, CH), d1.reshape(NW, NCH, CH))


def kernel(x, W_mh, b_mh, Wg, W1, b1, W2, b2, W_merge, b_merge):
    xm = x.reshape(S, D)
    y = _matmul_bias(xm, W_mh, b_mh)
    y16 = y.reshape(T, HD)
    e0, e1, g0, g1 = _router(y16, Wg)
    d0, d1, be = _sortmeta(e0.reshape(T), e1.reshape(T))
    d0 = d0.reshape(T)
    d1 = d1.reshape(T)
    yg, gyg = _dispatch(y16, d0, d1, g0.reshape(T), g1.reshape(T))
    eo = _grouped_ffn(be, yg, gyg, W1, b1, W2, b2)
    ymoe = _combine(eo, d0, d1)
    out = _matmul_bias(ymoe.reshape(S, D), W_merge, b_merge)
    return out.reshape(B, S, D)


# speedup vs baseline: 2.0629x; 2.0629x over previous
import functools

import jax
import jax.numpy as jnp
from jax import lax
from jax.experimental import pallas as pl
from jax.experimental.pallas import tpu as pltpu
from jax.experimental.pallas import tpu_sc as plsc

B = 1
S = 2048
D = 1024
H = 8
HD = D // H
T = S * H
E = 8
K = 2
F = 512

BM = 512
NBS = T * K // BM + E
PS = NBS * BM
NW = 32
TPW = T // NW
CH = 128
NCH = TPW // CH

_GELU_C = 0.7978845608028654


def _gelu(x):
    z = _GELU_C * x * (1.0 + 0.044715 * x * x)
    return x / (1.0 + jnp.exp(-2.0 * z))


def _mh_kernel(x_ref, w_ref, b_ref, o_ref):
    y = (
        jnp.dot(x_ref[...], w_ref[...], preferred_element_type=jnp.float32)
        + b_ref[...]
    )
    o_ref[...] = y.reshape(o_ref.shape)


def _mh_proj(x, w, b, bm=512):
    return pl.pallas_call(
        _mh_kernel,
        grid=(S // bm,),
        in_specs=[
            pl.BlockSpec((bm, D), lambda i: (i, 0)),
            pl.BlockSpec((D, D), lambda i: (0, 0)),
            pl.BlockSpec((1, D), lambda i: (0, 0)),
        ],
        out_specs=pl.BlockSpec((bm * H, HD), lambda i: (i, 0)),
        out_shape=jax.ShapeDtypeStruct((T, HD), jnp.float32),
    )(x, w, b.reshape(1, D))


def _router_kernel(y_ref, wg_ref, e0_ref, e1_ref, g0_ref, g1_ref):
    y = y_ref[...]
    logits = jnp.dot(y, wg_ref[...], preferred_element_type=jnp.float32)
    iota = jax.lax.broadcasted_iota(jnp.int32, logits.shape, 1)
    m1 = jnp.max(logits, axis=-1, keepdims=True)
    i1 = jnp.min(jnp.where(logits == m1, iota, E), axis=-1, keepdims=True)
    lm = jnp.where(iota == i1, -jnp.inf, logits)
    m2 = jnp.max(lm, axis=-1, keepdims=True)
    i2 = jnp.min(jnp.where(lm == m2, iota, E), axis=-1, keepdims=True)
    t = jnp.exp(m2 - m1)
    g0 = 1.0 / (1.0 + t)
    g1 = t * g0
    bm = y_ref.shape[0]
    e0_ref[...] = i1.reshape(bm // 128, 128)
    e1_ref[...] = i2.reshape(bm // 128, 128)
    g0_ref[...] = g0.reshape(bm // H, H)
    g1_ref[...] = g1.reshape(bm // H, H)


def _router(y16, Wg, bm=4096):
    nb = T // bm
    return pl.pallas_call(
        _router_kernel,
        grid=(nb,),
        in_specs=[
            pl.BlockSpec((bm, HD), lambda i: (i, 0)),
            pl.BlockSpec((HD, E), lambda i: (0, 0)),
        ],
        out_specs=[
            pl.BlockSpec((bm // 128, 128), lambda i: (i, 0)),
            pl.BlockSpec((bm // 128, 128), lambda i: (i, 0)),
            pl.BlockSpec((bm // H, H), lambda i: (i, 0)),
            pl.BlockSpec((bm // H, H), lambda i: (i, 0)),
        ],
        out_shape=[
            jax.ShapeDtypeStruct((T // 128, 128), jnp.int32),
            jax.ShapeDtypeStruct((T // 128, 128), jnp.int32),
            jax.ShapeDtypeStruct((S, E), jnp.float32),
            jax.ShapeDtypeStruct((S, E), jnp.float32),
        ],
    )(y16, Wg)


def _sortmeta_kernel(e0_ref, e1_ref, d0_ref, d1_ref, be_ref):
    f32 = jnp.float32
    ir = jax.lax.broadcasted_iota(jnp.int32, (128, 128), 0)
    ic = jax.lax.broadcasted_iota(jnp.int32, (128, 128), 1)
    U = (ir < ic).astype(f32)
    lr = jax.lax.broadcasted_iota(jnp.int32, (16, 16), 0)
    lc = jax.lax.broadcasted_iota(jnp.int32, (16, 16), 1)
    L16 = (lr > lc).astype(f32)

    e0 = e0_ref[...]
    e1 = e1_ref[...]

    counts = []
    for e in range(E):
        m = (e0 == e).astype(f32) + (e1 == e).astype(f32)
        counts.append(jnp.sum(m))
    starts = []
    s = jnp.float32(0.0)
    for e in range(E):
        starts.append(s)
        s = s + jnp.ceil(counts[e] / BM) * BM

    ib = jax.lax.broadcasted_iota(jnp.int32, (1, 256), 1).astype(f32) * BM
    be = jnp.zeros((1, 256), jnp.int32)
    for e in range(E):
        be = be + (ib >= starts[e]).astype(jnp.int32)
    be_ref[...] = be - 1

    carry = [jnp.float32(0.0)] * E
    for c in range(8):
        sl = slice(16 * c, 16 * c + 16)
        e0c = e0[sl, :]
        e1c = e1[sl, :]
        d0c = jnp.zeros((16, 128), f32)
        d1c = jnp.zeros((16, 128), f32)
        for e in range(E):
            m = (e0c == e).astype(f32) + (e1c == e).astype(f32)
            lane_excl = jnp.dot(m, U, preferred_element_type=f32)
            rowtot = jnp.sum(m, axis=1, keepdims=True)
            row_excl = jnp.dot(L16, rowtot, preferred_element_type=f32)
            slot = starts[e] + carry[e] + row_excl + lane_excl
            d0c = d0c + jnp.where(e0c == e, slot, 0.0)
            d1c = d1c + jnp.where(e1c == e, slot, 0.0)
            carry[e] = carry[e] + jnp.sum(m)
        d0_ref[sl, :] = d0c.astype(jnp.int32)
        d1_ref[sl, :] = d1c.astype(jnp.int32)


def _sortmeta(e0, e1):
    return pl.pallas_call(
        _sortmeta_kernel,
        grid=(1,),
        in_specs=[
            pl.BlockSpec((128, 128), lambda i: (0, 0)),
            pl.BlockSpec((128, 128), lambda i: (0, 0)),
        ],
        out_specs=[
            pl.BlockSpec((128, 128), lambda i: (0, 0)),
            pl.BlockSpec((128, 128), lambda i: (0, 0)),
            pl.BlockSpec((1, 256), lambda i: (0, 0)),
        ],
        out_shape=[
            jax.ShapeDtypeStruct((128, 128), jnp.int32),
            jax.ShapeDtypeStruct((128, 128), jnp.int32),
            jax.ShapeDtypeStruct((1, 256), jnp.int32),
        ],
    )(e0, e1)


def _dispatch_body(y_hbm, d0_hbm, d1_hbm, yg_hbm, ybuf, d0b, d1b, sem):
    wid = lax.axis_index("s") * 2 + lax.axis_index("c")
    base = wid * TPW
    pltpu.sync_copy(d0_hbm.at[pl.ds(wid * NCH, NCH)], d0b)
    pltpu.sync_copy(d1_hbm.at[pl.ds(wid * NCH, NCH)], d1b)
    pltpu.sync_copy(y_hbm.at[pl.ds(base, TPW)], ybuf)
    cps = []
    for j in range(NCH):
        rows = ybuf.at[pl.ds(j * CH, CH)]
        cps.append(pltpu.async_copy(rows, yg_hbm.at[d0b.at[j]], sem))
        cps.append(pltpu.async_copy(rows, yg_hbm.at[d1b.at[j]], sem))
    for cp in cps:
        cp.wait()


def _dispatch(y16, d0, d1):
    mesh = plsc.VectorSubcoreMesh(core_axis_name="c", subcore_axis_name="s")
    kfn = functools.partial(
        pl.kernel,
        out_type=jax.ShapeDtypeStruct((PS, HD), jnp.float32),
        mesh=mesh,
        scratch_types=[
            pltpu.VMEM((TPW, HD), jnp.float32),
            pltpu.VMEM((NCH, CH), jnp.int32),
            pltpu.VMEM((NCH, CH), jnp.int32),
            pltpu.SemaphoreType.DMA,
        ],
    )(_dispatch_body)
    return kfn(y16, d0, d1)


def _ffn_kernel(be_ref, yg_ref, w1_ref, b1_ref, w2_ref, b2_ref, o_ref):
    ygb = yg_ref[...]
    h = _gelu(
        jnp.dot(ygb, w1_ref[0], preferred_element_type=jnp.float32)
        + b1_ref[0]
    )
    o_ref[...] = (
        jnp.dot(h, w2_ref[0], preferred_element_type=jnp.float32) + b2_ref[0]
    )


def _grouped_ffn(be, yg, W1, b1, W2, b2):
    grid_spec = pltpu.PrefetchScalarGridSpec(
        num_scalar_prefetch=1,
        grid=(NBS,),
        in_specs=[
            pl.BlockSpec((BM, HD), lambda i, be: (i, 0)),
            pl.BlockSpec((1, HD, F), lambda i, be: (be[0, i], 0, 0)),
            pl.BlockSpec((1, 1, F), lambda i, be: (be[0, i], 0, 0)),
            pl.BlockSpec((1, F, HD), lambda i, be: (be[0, i], 0, 0)),
            pl.BlockSpec((1, 1, HD), lambda i, be: (be[0, i], 0, 0)),
        ],
        out_specs=pl.BlockSpec((BM, HD), lambda i, be: (i, 0)),
    )
    return pl.pallas_call(
        _ffn_kernel,
        grid_spec=grid_spec,
        out_shape=jax.ShapeDtypeStruct((PS, HD), jnp.float32),
    )(be, yg, W1, b1.reshape(E, 1, F), W2, b2.reshape(E, 1, HD))


def _combine_body(eo_hbm, d0_hbm, d1_hbm, r0_hbm, r1_hbm,
                  d0b, d1b, r0buf, r1buf, sem, wsem):
    wid = lax.axis_index("s") * 2 + lax.axis_index("c")
    base = wid * TPW
    pltpu.sync_copy(d0_hbm.at[pl.ds(wid * NCH, NCH)], d0b)
    pltpu.sync_copy(d1_hbm.at[pl.ds(wid * NCH, NCH)], d1b)
    half = NCH // 2
    for r in range(2):
        cps = []
        for jj in range(half):
            j = r * half + jj
            dst = pl.ds(jj * CH, CH)
            cps.append(pltpu.async_copy(eo_hbm.at[d0b.at[j]], r0buf.at[dst], sem))
            cps.append(pltpu.async_copy(eo_hbm.at[d1b.at[j]], r1buf.at[dst], sem))
        for cp in cps:
            cp.wait()
        out_sl = pl.ds(base + r * half * CH, half * CH)
        w0 = pltpu.async_copy(r0buf, r0_hbm.at[out_sl], wsem)
        w1 = pltpu.async_copy(r1buf, r1_hbm.at[out_sl], wsem)
        w0.wait()
        w1.wait()


def _combine(eo, d0, d1):
    mesh = plsc.VectorSubcoreMesh(core_axis_name="c", subcore_axis_name="s")
    kfn = functools.partial(
        pl.kernel,
        out_type=[
            jax.ShapeDtypeStruct((T, HD), jnp.float32),
            jax.ShapeDtypeStruct((T, HD), jnp.float32),
        ],
        mesh=mesh,
        scratch_types=[
            pltpu.VMEM((NCH, CH), jnp.int32),
            pltpu.VMEM((NCH, CH), jnp.int32),
            pltpu.VMEM((TPW // 2, HD), jnp.float32),
            pltpu.VMEM((TPW // 2, HD), jnp.float32),
            pltpu.SemaphoreType.DMA,
            pltpu.SemaphoreType.DMA,
        ],
    )(_combine_body)
    return kfn(eo, d0, d1)


def _merge_kernel(r0_ref, r1_ref, g0_ref, g1_ref, w_ref, b_ref, o_ref):
    bm = g0_ref.shape[0]
    ih = jax.lax.broadcasted_iota(jnp.int32, (E, D), 0)
    ij = jax.lax.broadcasted_iota(jnp.int32, (E, D), 1)
    expand = (ij // HD == ih).astype(jnp.float32)
    g0w = jnp.dot(g0_ref[...], expand, preferred_element_type=jnp.float32)
    g1w = jnp.dot(g1_ref[...], expand, preferred_element_type=jnp.float32)
    r0 = r0_ref[...].reshape(bm, D)
    r1 = r1_ref[...].reshape(bm, D)
    ym = g0w * r0 + g1w * r1
    o_ref[...] = (
        jnp.dot(ym, w_ref[...], preferred_element_type=jnp.float32)
        + b_ref[...]
    )


def _merge(r0, r1, g0, g1, w, b, bm=512):
    return pl.pallas_call(
        _merge_kernel,
        grid=(S // bm,),
        in_specs=[
            pl.BlockSpec((bm * H, HD), lambda i: (i, 0)),
            pl.BlockSpec((bm * H, HD), lambda i: (i, 0)),
            pl.BlockSpec((bm, E), lambda i: (i, 0)),
            pl.BlockSpec((bm, E), lambda i: (i, 0)),
            pl.BlockSpec((D, D), lambda i: (0, 0)),
            pl.BlockSpec((1, D), lambda i: (0, 0)),
        ],
        out_specs=pl.BlockSpec((bm, D), lambda i: (i, 0)),
        out_shape=jax.ShapeDtypeStruct((S, D), jnp.float32),
    )(r0, r1, g0, g1, w, b.reshape(1, D))


def kernel(x, W_mh, b_mh, Wg, W1, b1, W2, b2, W_merge, b_merge):
    xm = x.reshape(S, D)
    y16 = _mh_proj(xm, W_mh, b_mh)
    e0, e1, g0, g1 = _router(y16, Wg)
    d0, d1, be = _sortmeta(e0, e1)
    yg = _dispatch(y16, d0, d1)
    eo = _grouped_ffn(be, yg, W1, b1, W2, b2)
    r0, r1 = _combine(eo, d0, d1)
    out = _merge(r0, r1, g0, g1, W_merge, b_merge)
    return out.reshape(B, S, D)
